# baseline (device time: 11057 ns/iter reference)
import jax
import jax.numpy as jnp
from jax import lax
from jax.experimental import pallas as pl
from jax.experimental.pallas import tpu as pltpu

N_DEV = 4


def kernel(x):
    m_per, n = x.shape
    m_half = m_per // 2

    def body(x_ref, out_ref, send_sems, recv_sems, copy_sem):
        my_pos = lax.axis_index("i")
        left = (my_pos - 1) % N_DEV
        right = (my_pos + 1) % N_DEV
        opp = (my_pos + 2) % N_DEV

        def rdma(src, dst, s_slot, r_slot, target):
            return pltpu.make_async_remote_copy(
                src_ref=src,
                dst_ref=dst,
                send_sem=send_sems.at[s_slot],
                recv_sem=recv_sems.at[r_slot],
                device_id=(target,),
                device_id_type=pl.DeviceIdType.MESH,
            )

        local_copy = pltpu.make_async_copy(
            x_ref, out_ref.at[pl.ds(my_pos * m_per, m_per), :], copy_sem
        )
        local_copy.start()

        barrier_sem = pltpu.get_barrier_semaphore()
        for nbr in [left, right]:
            pl.semaphore_signal(
                barrier_sem, inc=1,
                device_id=(nbr,), device_id_type=pl.DeviceIdType.MESH,
            )
        pl.semaphore_wait(barrier_sem, 2)

        my_top = pl.ds(my_pos * m_per, m_half)
        my_bot = pl.ds(my_pos * m_per + m_half, m_half)

        s_top_r = rdma(x_ref.at[pl.ds(0, m_half), :],
                       out_ref.at[my_top, :], 0, 0, right)
        s_top_r.start()
        s_bot_l = rdma(x_ref.at[pl.ds(m_half, m_half), :],
                       out_ref.at[my_bot, :], 2, 2, left)
        s_bot_l.start()
        s_bot_r = rdma(x_ref.at[pl.ds(m_half, m_half), :],
                       out_ref.at[my_bot, :], 1, 1, right)
        s_bot_r.start()
        s_top_l = rdma(x_ref.at[pl.ds(0, m_half), :],
                       out_ref.at[my_top, :], 3, 3, left)
        s_top_l.start()

        half_src = x_ref.at[pl.ds(0, m_half), :]

        l_top = pl.ds(left * m_per, m_half)
        r_bot = pl.ds(right * m_per + m_half, m_half)

        rdma(half_src, out_ref.at[l_top, :], 0, 0, left).wait_recv()
        fwd_r = rdma(out_ref.at[l_top, :], out_ref.at[l_top, :], 4, 4, right)
        fwd_r.start()

        rdma(half_src, out_ref.at[r_bot, :], 2, 2, right).wait_recv()
        fwd_l = rdma(out_ref.at[r_bot, :], out_ref.at[r_bot, :], 5, 5, left)
        fwd_l.start()

        rdma(half_src, out_ref.at[pl.ds(left * m_per + m_half, m_half), :],
             1, 1, left).wait_recv()
        rdma(half_src, out_ref.at[pl.ds(right * m_per, m_half), :],
             3, 3, right).wait_recv()

        rdma(half_src, out_ref.at[pl.ds(opp * m_per, m_half), :],
             4, 4, left).wait_recv()
        rdma(half_src, out_ref.at[pl.ds(opp * m_per + m_half, m_half), :],
             5, 5, right).wait_recv()

        s_top_r.wait_send()
        s_bot_l.wait_send()
        s_bot_r.wait_send()
        s_top_l.wait_send()
        fwd_r.wait_send()
        fwd_l.wait_send()
        local_copy.wait()

    return pl.pallas_call(
        body,
        out_shape=jax.ShapeDtypeStruct((N_DEV * m_per, n), x.dtype),
        in_specs=[pl.BlockSpec(memory_space=pltpu.VMEM)],
        out_specs=pl.BlockSpec(memory_space=pltpu.VMEM),
        scratch_shapes=[
            pltpu.SemaphoreType.DMA((6,)),
            pltpu.SemaphoreType.DMA((6,)),
            pltpu.SemaphoreType.DMA,
        ],
        compiler_params=pltpu.CompilerParams(collective_id=0),
    )(x)
